# baseline (speedup 1.0000x reference)
import functools

import jax
import jax.numpy as jnp
from jax import lax
from jax.experimental import pallas as pl
from jax.experimental.pallas import tpu as pltpu
from jax.experimental.pallas import tpu_sc as plsc

VOCAB = 100000
DIM = 64
BATCH = 4096
SEQ = 200

NC = 2
NS = 16
NW = NC * NS

NBT = BATCH // 128
NST = SEQ // 8
NDT = DIM // 8

NBUF = 2
G = SEQ // NBUF

_mesh = plsc.VectorSubcoreMesh(
    core_axis_name="c", subcore_axis_name="s", num_cores=NC, num_subcores=NS
)


@functools.partial(
    pl.kernel,
    out_type=jax.ShapeDtypeStruct((SEQ, NDT, NBT, 8, 128), jnp.float32),
    mesh=_mesh,
    scratch_types=[
        pltpu.VMEM((NST, 8, 128), jnp.int32),
        pltpu.VMEM((NBUF, 128, DIM), jnp.float32),
        pltpu.VMEM((NBUF, NDT, 8, 129), jnp.float32),
        pltpu.SemaphoreType.DMA,
        pltpu.SemaphoreType.DMA,
        pltpu.SemaphoreType.DMA,
    ],
    compiler_params=pltpu.CompilerParams(
        use_tc_tiling_on_sc=False, needs_layout_passes=False
    ),
)
def _sc_gather(xq_hbm, tab_hbm, out_hbm, idx_all, rows_v, trans_v,
               sem_i, sem_g, sem_w):
    wid = lax.axis_index("s") * NC + lax.axis_index("c")

    def gather_cp(b, st, sr):
        return pltpu.make_async_copy(
            tab_hbm.at[idx_all.at[st, sr]], rows_v.at[b], sem_g
        )

    def wout_cp(b, s, dt):
        return pltpu.make_async_copy(
            trans_v.at[b, dt, :, pl.ds(0, 128)],
            out_hbm.at[s, dt, wid],
            sem_w,
        )

    for st in range(NST):
        pltpu.async_copy(xq_hbm.at[st, wid], idx_all.at[st], sem_i)
    for st in range(NST):
        pltpu.make_async_copy(xq_hbm.at[st, wid], idx_all.at[st], sem_i).wait()

    for b in range(NBUF):
        gather_cp(b, 0, b).start()

    iota = lax.iota(jnp.int32, 16)
    iota8 = lax.div(iota, 8)
    dr8 = lax.rem(iota, 8)
    dts = [iota8 + 2 * c for c in range(DIM // 16)]

    def outer(q, carry):
        st_lo = q // 4
        sr_base = (q % 4) * NBUF
        st_hi = (q + 1) // 4
        sr_hi_base = ((q + 1) % 4) * NBUF
        izero = lax.shift_right_logical(q, 31)
        for b in range(NBUF):
            s = q * NBUF + b
            gather_cp(b, st_lo, sr_base + b).wait()

            @pl.when(q > 0)
            def _():
                for dt in range(NDT):
                    wout_cp(b, s, dt).wait()

            for i2 in range(64):
                vs = []
                for k in range(2):
                    i = i2 * 2 + k
                    for c in range(DIM // 16):
                        vs.append((i, c, rows_v[b, i, pl.ds(c * 16, 16)]))
                for i, c, v in vs:
                    ivec = lax.broadcast(izero + i, (16,))
                    plsc.store_scatter(
                        trans_v.at[b], [dts[c], dr8, ivec], v
                    )
            @pl.when(q < G - 1)
            def _():
                gather_cp(b, st_hi, sr_hi_base + b).start()

            for dt in range(NDT):
                wout_cp(b, s, dt).start()

        return carry

    lax.fori_loop(0, G, outer, 0)

    for b in range(NBUF):
        for dt in range(NDT):
            wout_cp(b, (G - 1) * NBUF + b, dt).wait()


def _mask_body(x_ref, o_ref):
    o_ref[...] = x_ref[...] != 0


_tc_mask = pl.pallas_call(
    _mask_body,
    out_shape=jax.ShapeDtypeStruct((BATCH, SEQ), jnp.bool_),
    grid=(BATCH // 512,),
    in_specs=[pl.BlockSpec((512, SEQ), lambda i: (i, 0))],
    out_specs=pl.BlockSpec((512, SEQ), lambda i: (i, 0)),
)


def kernel(x, table, concepts):
    ext = jnp.concatenate([table, concepts], axis=0)
    xq = x.reshape(NBT, 128, NST, 8).transpose(2, 0, 3, 1)
    a5 = _sc_gather(xq, ext)
    embeds = a5.transpose(2, 4, 0, 1, 3).reshape(BATCH, SEQ, DIM)
    mask = _tc_mask(x)
    return embeds, mask


# speedup vs baseline: 1.0428x; 1.0428x over previous
import functools

import jax
import jax.numpy as jnp
from jax import lax
from jax.experimental import pallas as pl
from jax.experimental.pallas import tpu as pltpu
from jax.experimental.pallas import tpu_sc as plsc

VOCAB = 100000
DIM = 64
BATCH = 4096
SEQ = 200

NC = 2
NS = 16
NW = NC * NS

NBT = BATCH // 128
NST = SEQ // 8
NDT = DIM // 8

NBUF = 2
G = SEQ // NBUF

_mesh = plsc.VectorSubcoreMesh(
    core_axis_name="c", subcore_axis_name="s", num_cores=NC, num_subcores=NS
)


@functools.partial(
    pl.kernel,
    out_type=jax.ShapeDtypeStruct((SEQ, NDT, NBT, 8, 128), jnp.float32),
    mesh=_mesh,
    scratch_types=[
        pltpu.VMEM((NST, 8, 128), jnp.int32),
        pltpu.VMEM((2 * NBUF, 128, DIM), jnp.float32),
        pltpu.VMEM((NBUF, NDT, 8, 129), jnp.float32),
        pltpu.SemaphoreType.DMA,
        pltpu.SemaphoreType.DMA,
        pltpu.SemaphoreType.DMA,
    ],
    compiler_params=pltpu.CompilerParams(
        use_tc_tiling_on_sc=False, needs_layout_passes=False
    ),
)
def _sc_gather(xq_hbm, tab_hbm, out_hbm, idx_all, rows_v, trans_v,
               sem_i, sem_g, sem_w):
    wid = lax.axis_index("s") * NC + lax.axis_index("c")

    def gather_cp(rb, st, sr):
        return pltpu.make_async_copy(
            tab_hbm.at[idx_all.at[st, sr]], rows_v.at[rb], sem_g
        )

    def wout_cp(b, s, dt):
        return pltpu.make_async_copy(
            trans_v.at[b, dt, :, pl.ds(0, 128)],
            out_hbm.at[s, dt, wid],
            sem_w,
        )

    for st in range(NST):
        pltpu.async_copy(xq_hbm.at[st, wid], idx_all.at[st], sem_i)
    for st in range(NST):
        pltpu.make_async_copy(xq_hbm.at[st, wid], idx_all.at[st], sem_i).wait()

    for b in range(NBUF):
        gather_cp(b, 0, b).start()

    iota = lax.iota(jnp.int32, 16)
    iota8 = lax.div(iota, 8)
    dr8 = lax.rem(iota, 8)
    dts = [iota8 + 2 * c for c in range(DIM // 16)]

    def outer(q, carry):
        st_lo = q // 4
        sr_base = (q % 4) * NBUF
        st_hi = (q + 1) // 4
        sr_hi_base = ((q + 1) % 4) * NBUF
        izero = lax.shift_right_logical(q, 31)
        rb_q = (q % 2) * NBUF
        rb2_q = ((q + 1) % 2) * NBUF
        for b in range(NBUF):
            s = q * NBUF + b
            rb = rb_q + b
            gather_cp(rb, st_lo, sr_base + b).wait()

            @pl.when(q > 0)
            def _():
                for dt in range(NDT):
                    wout_cp(b, s, dt).wait()

            @pl.when(q < G - 1)
            def _():
                gather_cp(rb2_q + b, st_hi, sr_hi_base + b).start()

            for c in range(DIM // 16):
                for i8 in range(16):
                    vs = []
                    for k in range(8):
                        i = i8 * 8 + k
                        vs.append((i, rows_v[rb, i, pl.ds(c * 16, 16)]))
                    for i, v in vs:
                        ivec = lax.broadcast(izero + i, (16,))
                        plsc.store_scatter(
                            trans_v.at[b], [dts[c], dr8, ivec], v
                        )
                wout_cp(b, s, 2 * c).start()
                wout_cp(b, s, 2 * c + 1).start()

        return carry

    lax.fori_loop(0, G, outer, 0)

    for b in range(NBUF):
        for dt in range(NDT):
            wout_cp(b, (G - 1) * NBUF + b, dt).wait()


def _mask_body(x_ref, o_ref):
    o_ref[...] = x_ref[...] != 0


_tc_mask = pl.pallas_call(
    _mask_body,
    out_shape=jax.ShapeDtypeStruct((BATCH, SEQ), jnp.bool_),
    grid=(BATCH // 512,),
    in_specs=[pl.BlockSpec((512, SEQ), lambda i: (i, 0))],
    out_specs=pl.BlockSpec((512, SEQ), lambda i: (i, 0)),
)


def kernel(x, table, concepts):
    ext = jnp.concatenate([table, concepts], axis=0)
    xq = x.reshape(NBT, 128, NST, 8).transpose(2, 0, 3, 1)
    a5 = _sc_gather(xq, ext)
    embeds = a5.transpose(2, 4, 0, 1, 3).reshape(BATCH, SEQ, DIM)
    mask = _tc_mask(x)
    return embeds, mask
